# baseline (speedup 1.0000x reference)
import functools

import jax
import jax.numpy as jnp
from jax import lax
from jax.experimental import pallas as pl
from jax.experimental.pallas import tpu as pltpu
from jax.experimental.pallas import tpu_sc as plsc

N = 10000
F = 32
H = 64
G = 100
S = 3
OUT = 64
E = 640000

NC = 2
NS = 16
NW = NC * NS
CHUNK = 128
CHUNKS_PER_W = 160
UNROLL = 8
EW = CHUNK * CHUNKS_PER_W
EP = EW * NW
AR = 10240
RPT = AR // NS
GB = 512


def _gru_body(x_ref, bf_ref, ph_ref, wih_ref, whh_ref, bih_ref, bhh_ref,
              w1_ref, out_ref):
    B = x_ref.shape[0]
    xb = x_ref[...]
    bf = bf_ref[...]
    ph = ph_ref[...]
    wih = wih_ref[...]
    whh = whh_ref[...]
    bih = bih_ref[...]
    bhh = bhh_ref[...]
    h = jnp.broadcast_to(jnp.cos(ph), (B, F))
    xe = jnp.cos(xb[:, :, None] * bf[None, :, :] + ph[None, :, :])
    gi_all = jnp.dot(xe.reshape(B * F, F), wih,
                     preferred_element_type=jnp.float32) + bih
    gi_all = gi_all.reshape(B, F, 3 * F)
    for t in range(F):
        gi = gi_all[:, t, :]
        gh = jnp.dot(h, whh, preferred_element_type=jnp.float32) + bhh
        r = jax.nn.sigmoid(gi[:, 0:F] + gh[:, 0:F])
        z = jax.nn.sigmoid(gi[:, F:2 * F] + gh[:, F:2 * F])
        ng = jnp.tanh(gi[:, 2 * F:] + r * gh[:, 2 * F:])
        h = (1.0 - z) * ng + z * h
    out_ref[...] = jnp.dot(h, w1_ref[...], preferred_element_type=jnp.float32)


def _gru_call(x, bf2, ph2, wihT, whhT, bih2, bhh2, W1):
    B = 1000
    grid = N // B
    return pl.pallas_call(
        _gru_body,
        grid=(grid,),
        in_specs=[
            pl.BlockSpec((B, F), lambda i: (i, 0)),
            pl.BlockSpec((1, F), lambda i: (0, 0)),
            pl.BlockSpec((1, F), lambda i: (0, 0)),
            pl.BlockSpec((F, 3 * F), lambda i: (0, 0)),
            pl.BlockSpec((F, 3 * F), lambda i: (0, 0)),
            pl.BlockSpec((1, 3 * F), lambda i: (0, 0)),
            pl.BlockSpec((1, 3 * F), lambda i: (0, 0)),
            pl.BlockSpec((F, H), lambda i: (0, 0)),
        ],
        out_specs=pl.BlockSpec((B, H), lambda i: (i, 0)),
        out_shape=jax.ShapeDtypeStruct((N, H), jnp.float32),
    )(x, bf2, ph2, wihT, whhT, bih2, bhh2, W1)


def _deg_body(dst_hbm, ones_hbm, zeros_hbm, out_hbm, onesb, zbuf, didx, acc,
              sem):
    c = lax.axis_index("c")
    s = lax.axis_index("s")
    wid = c * NS + s

    pltpu.sync_copy(zeros_hbm, zbuf)
    pltpu.sync_copy(zbuf, acc.at[pl.ds(s * RPT, RPT)])
    pltpu.sync_copy(ones_hbm, onesb)
    pltpu.sync_copy(dst_hbm.at[pl.ds(wid * EW, EW)], didx)
    plsc.subcore_barrier()

    def chunk(i, carry):
        pltpu.sync_copy(onesb, acc.at[didx.at[pl.ds(i * CHUNK, CHUNK)]], add=True)
        return carry

    lax.fori_loop(0, CHUNKS_PER_W, chunk, 0)
    plsc.subcore_barrier()
    pltpu.sync_copy(acc.at[pl.ds(s * RPT, RPT)], zbuf)
    pltpu.sync_copy(zbuf, out_hbm.at[pl.ds(c * AR + s * RPT, RPT)])


def _deg_call(dstp, ones16, zeros16):
    mesh = plsc.VectorSubcoreMesh(core_axis_name="c", subcore_axis_name="s")
    kfn = pl.kernel(
        _deg_body,
        mesh=mesh,
        out_type=jax.ShapeDtypeStruct((NC * AR, 16), jnp.float32),
        scratch_types=[
            pltpu.VMEM((CHUNK, 16), jnp.float32),
            pltpu.VMEM((RPT, 16), jnp.float32),
            pltpu.VMEM((EW,), jnp.int32),
            pltpu.VMEM_SHARED((AR, 16), jnp.float32),
            pltpu.SemaphoreType.DMA,
        ],
        compiler_params=pltpu.CompilerParams(use_tc_tiling_on_sc=False),
    )
    return kfn(dstp, ones16, zeros16)


def _edge_body(table_hbm, src_hbm, dst_hbm, zeros_hbm, out_hbm,
               sidx, didx, rows0, rows1, obuf, acc, sem0, sem1):
    c = lax.axis_index("c")
    s = lax.axis_index("s")
    wid = c * NS + s

    BLKE = UNROLL * CHUNK
    NBLK = CHUNKS_PER_W // UNROLL
    pltpu.sync_copy(zeros_hbm, obuf)
    pltpu.sync_copy(obuf, acc.at[pl.ds(s * RPT, RPT)])
    base = wid * EW
    plsc.subcore_barrier()

    def blk(i, carry):
        pltpu.sync_copy(src_hbm.at[pl.ds(base + i * BLKE, BLKE)], sidx)
        pltpu.sync_copy(dst_hbm.at[pl.ds(base + i * BLKE, BLKE)], didx)
        for j in range(UNROLL):
            b, bs = (rows0, sem0) if j % 2 == 0 else (rows1, sem1)
            pltpu.async_copy(
                table_hbm.at[sidx.at[pl.ds(j * CHUNK, CHUNK)]], b, bs).wait()
            pltpu.sync_copy(
                b, acc.at[didx.at[pl.ds(j * CHUNK, CHUNK)]], add=True)
        return carry

    lax.fori_loop(0, NBLK, blk, 0)
    plsc.subcore_barrier()
    pltpu.sync_copy(acc.at[pl.ds(s * RPT, RPT)], obuf)
    pltpu.sync_copy(obuf, out_hbm.at[pl.ds(c * AR + s * RPT, RPT)])


def _edge_call(table, srcp, dstp, zeros64):
    mesh = plsc.VectorSubcoreMesh(core_axis_name="c", subcore_axis_name="s")
    kfn = pl.kernel(
        _edge_body,
        mesh=mesh,
        out_type=jax.ShapeDtypeStruct((NC * AR, H), jnp.float32),
        scratch_types=[
            pltpu.VMEM((UNROLL * CHUNK,), jnp.int32),
            pltpu.VMEM((UNROLL * CHUNK,), jnp.int32),
            pltpu.VMEM((CHUNK, H), jnp.float32),
            pltpu.VMEM((CHUNK, H), jnp.float32),
            pltpu.VMEM((RPT, H), jnp.float32),
            pltpu.VMEM_SHARED((AR, H), jnp.float32),
            pltpu.SemaphoreType.DMA,
            pltpu.SemaphoreType.DMA,
        ],
        compiler_params=pltpu.CompilerParams(use_tc_tiling_on_sc=False),
    )
    return kfn(table, srcp, dstp, zeros64)


def _rgather_body(table_hbm, idx_hbm, out_hbm, idxb, rows, sem):
    c = lax.axis_index("c")
    s = lax.axis_index("s")
    wid = c * NS + s
    k = GB // NW
    pltpu.sync_copy(idx_hbm.at[pl.ds(wid * k, k)], idxb)
    pltpu.async_copy(table_hbm.at[idxb], rows, sem).wait()
    pltpu.sync_copy(rows, out_hbm.at[pl.ds(wid * k, k)])


def _rgather_call(h2, sibp):
    mesh = plsc.VectorSubcoreMesh(core_axis_name="c", subcore_axis_name="s")
    kfn = pl.kernel(
        _rgather_body,
        mesh=mesh,
        out_type=jax.ShapeDtypeStruct((GB, H), jnp.float32),
        scratch_types=[
            pltpu.VMEM((GB // NW,), jnp.int32),
            pltpu.VMEM((GB // NW, H), jnp.float32),
            pltpu.SemaphoreType.DMA,
        ],
        compiler_params=pltpu.CompilerParams(use_tc_tiling_on_sc=False),
    )
    return kfn(h2, sibp)


def _scale_body(d0_ref, d1_ref, g1_ref, dis_ref, hws_ref):
    deg = 1.0 + d0_ref[:, 0:1] + d1_ref[:, 0:1]
    dis = lax.rsqrt(deg)
    dis_ref[...] = dis
    hws_ref[...] = g1_ref[...] * dis


def _scale_call(d0, d1, g1):
    B = 1000
    grid = N // B
    return pl.pallas_call(
        _scale_body,
        grid=(grid,),
        in_specs=[
            pl.BlockSpec((B, 16), lambda i: (i, 0)),
            pl.BlockSpec((B, 16), lambda i: (i, 0)),
            pl.BlockSpec((B, H), lambda i: (i, 0)),
        ],
        out_specs=[
            pl.BlockSpec((B, 1), lambda i: (i, 0)),
            pl.BlockSpec((B, H), lambda i: (i, 0)),
        ],
        out_shape=[
            jax.ShapeDtypeStruct((N, 1), jnp.float32),
            jax.ShapeDtypeStruct((N, H), jnp.float32),
        ],
    )(d0, d1, g1)


def _mid_body(a0_ref, a1_ref, hws_ref, dis_ref, b_ref, w_ref, out_ref):
    dis = dis_ref[...]
    h1 = jax.nn.relu((a0_ref[...] + a1_ref[...] + hws_ref[...]) * dis
                     + b_ref[...])
    out_ref[...] = jnp.dot(h1, w_ref[...],
                           preferred_element_type=jnp.float32) * dis


def _mid_call(a0, a1, hws1, dis, b1, W2):
    B = 1000
    grid = N // B
    return pl.pallas_call(
        _mid_body,
        grid=(grid,),
        in_specs=[
            pl.BlockSpec((B, H), lambda i: (i, 0)),
            pl.BlockSpec((B, H), lambda i: (i, 0)),
            pl.BlockSpec((B, H), lambda i: (i, 0)),
            pl.BlockSpec((B, 1), lambda i: (i, 0)),
            pl.BlockSpec((1, H), lambda i: (0, 0)),
            pl.BlockSpec((H, H), lambda i: (0, 0)),
        ],
        out_specs=pl.BlockSpec((B, H), lambda i: (i, 0)),
        out_shape=jax.ShapeDtypeStruct((N, H), jnp.float32),
    )(a0, a1, hws1, dis, b1, W2)


def _h2_body(a0_ref, a1_ref, hws_ref, dis_ref, b_ref, out_ref):
    out_ref[...] = jax.nn.relu(
        (a0_ref[...] + a1_ref[...] + hws_ref[...]) * dis_ref[...] + b_ref[...])


def _h2_call(a0, a1, hws2, dis, b2):
    B = 1000
    grid = N // B
    return pl.pallas_call(
        _h2_body,
        grid=(grid,),
        in_specs=[
            pl.BlockSpec((B, H), lambda i: (i, 0)),
            pl.BlockSpec((B, H), lambda i: (i, 0)),
            pl.BlockSpec((B, H), lambda i: (i, 0)),
            pl.BlockSpec((B, 1), lambda i: (i, 0)),
            pl.BlockSpec((1, H), lambda i: (0, 0)),
        ],
        out_specs=pl.BlockSpec((B, H), lambda i: (i, 0)),
        out_shape=jax.ShapeDtypeStruct((N, H), jnp.float32),
    )(a0, a1, hws2, dis, b2)


def _ib_body(bm_ref, ib_ref):
    bm = bm_ref[...]
    gi = lax.broadcasted_iota(jnp.int32, (G, 80, 128), 0)
    m = (bm[None, :, :] < gi).astype(jnp.int32)
    cnt = jnp.sum(jnp.sum(m, axis=2), axis=1)
    ib_ref[...] = cnt[:, None]


def _ib_call(bm):
    return pl.pallas_call(
        _ib_body,
        out_shape=jax.ShapeDtypeStruct((G, 1), jnp.int32),
    )(bm)


def _final_body(mb_ref, wl_ref, bl_ref, out_ref):
    out_ref[...] = jnp.dot(mb_ref[...], wl_ref[...],
                           preferred_element_type=jnp.float32) + bl_ref[...]


def _final_call(mb, Wl, bl2):
    return pl.pallas_call(
        _final_body,
        out_shape=jax.ShapeDtypeStruct((G, OUT), jnp.float32),
    )(mb, Wl, bl2)


@jax.jit
def _run(x, edge_index, set_indice, batch_idx, basis_freq, phase,
         W_ih, W_hh, b_ih, b_hh, W1, b1, W2, b2, Wl, bl):
    f32 = jnp.float32
    i32 = jnp.int32
    src = edge_index[0]
    dst = edge_index[1]
    srcp = jnp.concatenate([src, jnp.zeros((EP - E,), i32)])
    dstp = jnp.concatenate([dst, jnp.full((EP - E,), N, i32)])
    zeros64 = jnp.zeros((RPT, H), f32)
    zeros16 = jnp.zeros((RPT, 16), f32)
    ones16 = jnp.ones((CHUNK, 16), f32)
    bf2 = basis_freq.reshape(1, F)
    ph2 = phase.reshape(1, F)
    wihT = W_ih.T
    whhT = W_hh.T
    bih2 = b_ih.reshape(1, 3 * F)
    bhh2 = b_hh.reshape(1, 3 * F)
    bm = jnp.concatenate([batch_idx,
                          jnp.full((AR - N,), G + 1, i32)]).reshape(80, 128)

    g1 = _gru_call(x, bf2, ph2, wihT, whhT, bih2, bhh2, W1)
    degf = _deg_call(dstp, ones16, zeros16)
    dis, hws1 = _scale_call(degf[:AR], degf[AR:], g1)
    acc1 = _edge_call(hws1, srcp, dstp, zeros64)
    hws2 = _mid_call(acc1[:AR][:N], acc1[AR:][:N], hws1, dis,
                     b1.reshape(1, H), W2)
    acc2 = _edge_call(hws2, srcp, dstp, zeros64)
    h2 = _h2_call(acc2[:AR][:N], acc2[AR:][:N], hws2, dis, b2.reshape(1, H))
    ib = _ib_call(bm)
    sib = jnp.clip(ib + set_indice, 0, N - 1).reshape(-1)
    sibp = jnp.concatenate([sib, jnp.zeros((GB - G * S,), i32)])
    gath = _rgather_call(h2, sibp)
    mb = gath[:G * S].reshape(G, S * H)
    return _final_call(mb, Wl, bl.reshape(1, OUT))


def kernel(x, edge_index, set_indice, batch_idx, num_graphs, basis_freq,
           phase, W_ih, W_hh, b_ih, b_hh, W1, b1, W2, b2, Wl, bl):
    return _run(x, edge_index, set_indice, batch_idx, basis_freq, phase,
                W_ih, W_hh, b_ih, b_hh, W1, b1, W2, b2, Wl, bl)


# speedup vs baseline: 1.3905x; 1.3905x over previous
import functools

import jax
import jax.numpy as jnp
from jax import lax
from jax.experimental import pallas as pl
from jax.experimental.pallas import tpu as pltpu
from jax.experimental.pallas import tpu_sc as plsc

N = 10000
F = 32
H = 64
G = 100
S = 3
OUT = 64
E = 640000

NC = 2
NS = 16
NW = NC * NS
CHUNK = 128
CHUNKS_PER_W = 160
UNROLL = 8
EW = CHUNK * CHUNKS_PER_W
EP = EW * NW
AR = 10240
RPT = AR // NS
GB = 512


def _gru_body(x_ref, bf_ref, ph_ref, wih_ref, whh_ref, bih_ref, bhh_ref,
              w1_ref, out_ref):
    B = x_ref.shape[0]
    xb = x_ref[...]
    bf = bf_ref[...]
    ph = ph_ref[...]
    wih = wih_ref[...]
    whh = whh_ref[...]
    bih = bih_ref[...]
    bhh = bhh_ref[...]
    h = jnp.broadcast_to(jnp.cos(ph), (B, F))
    for t in range(F):
        xt = jnp.cos(xb[:, t:t + 1] * bf + ph)
        gi = jnp.dot(xt, wih, preferred_element_type=jnp.float32) + bih
        gh = jnp.dot(h, whh, preferred_element_type=jnp.float32) + bhh
        r = jax.nn.sigmoid(gi[:, 0:F] + gh[:, 0:F])
        z = jax.nn.sigmoid(gi[:, F:2 * F] + gh[:, F:2 * F])
        ng = jnp.tanh(gi[:, 2 * F:] + r * gh[:, 2 * F:])
        h = (1.0 - z) * ng + z * h
    out_ref[...] = jnp.dot(h, w1_ref[...], preferred_element_type=jnp.float32)


def _gru_call(x, bf2, ph2, wihT, whhT, bih2, bhh2, W1):
    B = 2000
    grid = N // B
    return pl.pallas_call(
        _gru_body,
        grid=(grid,),
        in_specs=[
            pl.BlockSpec((B, F), lambda i: (i, 0)),
            pl.BlockSpec((1, F), lambda i: (0, 0)),
            pl.BlockSpec((1, F), lambda i: (0, 0)),
            pl.BlockSpec((F, 3 * F), lambda i: (0, 0)),
            pl.BlockSpec((F, 3 * F), lambda i: (0, 0)),
            pl.BlockSpec((1, 3 * F), lambda i: (0, 0)),
            pl.BlockSpec((1, 3 * F), lambda i: (0, 0)),
            pl.BlockSpec((F, H), lambda i: (0, 0)),
        ],
        out_specs=pl.BlockSpec((B, H), lambda i: (i, 0)),
        out_shape=jax.ShapeDtypeStruct((N, H), jnp.float32),
    )(x, bf2, ph2, wihT, whhT, bih2, bhh2, W1)


def _deg_body(dst_hbm, ones_hbm, zeros_hbm, out_hbm, onesb, zbuf, didx, acc,
              sem):
    c = lax.axis_index("c")
    s = lax.axis_index("s")
    wid = c * NS + s

    pltpu.sync_copy(zeros_hbm, zbuf)
    pltpu.sync_copy(zbuf, acc.at[pl.ds(s * RPT, RPT)])
    pltpu.sync_copy(ones_hbm, onesb)
    plsc.subcore_barrier()
    base = wid * EW

    def chunk(i, carry):
        pltpu.sync_copy(dst_hbm.at[pl.ds(base + i * CHUNK, CHUNK)], didx)
        pltpu.sync_copy(onesb, acc.at[didx], add=True)
        return carry

    lax.fori_loop(0, CHUNKS_PER_W, chunk, 0)
    plsc.subcore_barrier()
    pltpu.sync_copy(acc.at[pl.ds(s * RPT, RPT)], zbuf)
    pltpu.sync_copy(zbuf, out_hbm.at[pl.ds(c * AR + s * RPT, RPT)])


def _deg_call(dstp, ones16, zeros16):
    mesh = plsc.VectorSubcoreMesh(core_axis_name="c", subcore_axis_name="s")
    kfn = pl.kernel(
        _deg_body,
        mesh=mesh,
        out_type=jax.ShapeDtypeStruct((NC * AR, 16), jnp.float32),
        scratch_types=[
            pltpu.VMEM((CHUNK, 16), jnp.float32),
            pltpu.VMEM((RPT, 16), jnp.float32),
            pltpu.VMEM((CHUNK,), jnp.int32),
            pltpu.VMEM_SHARED((AR, 16), jnp.float32),
            pltpu.SemaphoreType.DMA,
        ],
        compiler_params=pltpu.CompilerParams(use_tc_tiling_on_sc=False),
    )
    return kfn(dstp, ones16, zeros16)


def _edge_body(table_hbm, src_hbm, dst_hbm, zeros_hbm, out_hbm,
               sidx, didx, rows0, rows1, obuf, acc, sem0, sem1):
    c = lax.axis_index("c")
    s = lax.axis_index("s")
    wid = c * NS + s

    pltpu.sync_copy(zeros_hbm, obuf)
    pltpu.sync_copy(obuf, acc.at[pl.ds(s * RPT, RPT)])
    base = wid * EW
    plsc.subcore_barrier()

    def chunk(i, carry):
        off = base + i * CHUNK
        pltpu.sync_copy(src_hbm.at[pl.ds(off, CHUNK)], sidx)
        pltpu.sync_copy(dst_hbm.at[pl.ds(off, CHUNK)], didx)
        pltpu.async_copy(table_hbm.at[sidx], rows0, sem0).wait()
        pltpu.sync_copy(rows0, acc.at[didx], add=True)
        return carry

    lax.fori_loop(0, CHUNKS_PER_W, chunk, 0)
    plsc.subcore_barrier()
    pltpu.sync_copy(acc.at[pl.ds(s * RPT, RPT)], obuf)
    pltpu.sync_copy(obuf, out_hbm.at[pl.ds(c * AR + s * RPT, RPT)])


def _edge_call(table, srcp, dstp, zeros64):
    mesh = plsc.VectorSubcoreMesh(core_axis_name="c", subcore_axis_name="s")
    kfn = pl.kernel(
        _edge_body,
        mesh=mesh,
        out_type=jax.ShapeDtypeStruct((NC * AR, H), jnp.float32),
        scratch_types=[
            pltpu.VMEM((CHUNK,), jnp.int32),
            pltpu.VMEM((CHUNK,), jnp.int32),
            pltpu.VMEM((CHUNK, H), jnp.float32),
            pltpu.VMEM((CHUNK, H), jnp.float32),
            pltpu.VMEM((RPT, H), jnp.float32),
            pltpu.VMEM_SHARED((AR, H), jnp.float32),
            pltpu.SemaphoreType.DMA,
            pltpu.SemaphoreType.DMA,
        ],
        compiler_params=pltpu.CompilerParams(use_tc_tiling_on_sc=False),
    )
    return kfn(table, srcp, dstp, zeros64)


def _rgather_body(table_hbm, idx_hbm, out_hbm, idxb, rows, sem):
    c = lax.axis_index("c")
    s = lax.axis_index("s")
    wid = c * NS + s
    k = GB // NW
    pltpu.sync_copy(idx_hbm.at[pl.ds(wid * k, k)], idxb)
    pltpu.async_copy(table_hbm.at[idxb], rows, sem).wait()
    pltpu.sync_copy(rows, out_hbm.at[pl.ds(wid * k, k)])


def _rgather_call(h2, sibp):
    mesh = plsc.VectorSubcoreMesh(core_axis_name="c", subcore_axis_name="s")
    kfn = pl.kernel(
        _rgather_body,
        mesh=mesh,
        out_type=jax.ShapeDtypeStruct((GB, H), jnp.float32),
        scratch_types=[
            pltpu.VMEM((GB // NW,), jnp.int32),
            pltpu.VMEM((GB // NW, H), jnp.float32),
            pltpu.SemaphoreType.DMA,
        ],
        compiler_params=pltpu.CompilerParams(use_tc_tiling_on_sc=False),
    )
    return kfn(h2, sibp)


def _scale_body(d0_ref, d1_ref, g1_ref, dis_ref, hws_ref):
    deg = 1.0 + d0_ref[:, 0:1] + d1_ref[:, 0:1]
    dis = lax.rsqrt(deg)
    dis_ref[...] = dis
    hws_ref[...] = g1_ref[...] * dis


def _scale_call(d0, d1, g1):
    B = 1000
    grid = N // B
    return pl.pallas_call(
        _scale_body,
        grid=(grid,),
        in_specs=[
            pl.BlockSpec((B, 16), lambda i: (i, 0)),
            pl.BlockSpec((B, 16), lambda i: (i, 0)),
            pl.BlockSpec((B, H), lambda i: (i, 0)),
        ],
        out_specs=[
            pl.BlockSpec((B, 1), lambda i: (i, 0)),
            pl.BlockSpec((B, H), lambda i: (i, 0)),
        ],
        out_shape=[
            jax.ShapeDtypeStruct((N, 1), jnp.float32),
            jax.ShapeDtypeStruct((N, H), jnp.float32),
        ],
    )(d0, d1, g1)


def _mid_body(a0_ref, a1_ref, hws_ref, dis_ref, b_ref, w_ref, out_ref):
    dis = dis_ref[...]
    h1 = jax.nn.relu((a0_ref[...] + a1_ref[...] + hws_ref[...]) * dis
                     + b_ref[...])
    out_ref[...] = jnp.dot(h1, w_ref[...],
                           preferred_element_type=jnp.float32) * dis


def _mid_call(a0, a1, hws1, dis, b1, W2):
    B = 1000
    grid = N // B
    return pl.pallas_call(
        _mid_body,
        grid=(grid,),
        in_specs=[
            pl.BlockSpec((B, H), lambda i: (i, 0)),
            pl.BlockSpec((B, H), lambda i: (i, 0)),
            pl.BlockSpec((B, H), lambda i: (i, 0)),
            pl.BlockSpec((B, 1), lambda i: (i, 0)),
            pl.BlockSpec((1, H), lambda i: (0, 0)),
            pl.BlockSpec((H, H), lambda i: (0, 0)),
        ],
        out_specs=pl.BlockSpec((B, H), lambda i: (i, 0)),
        out_shape=jax.ShapeDtypeStruct((N, H), jnp.float32),
    )(a0, a1, hws1, dis, b1, W2)


def _h2_body(a0_ref, a1_ref, hws_ref, dis_ref, b_ref, out_ref):
    out_ref[...] = jax.nn.relu(
        (a0_ref[...] + a1_ref[...] + hws_ref[...]) * dis_ref[...] + b_ref[...])


def _h2_call(a0, a1, hws2, dis, b2):
    B = 1000
    grid = N // B
    return pl.pallas_call(
        _h2_body,
        grid=(grid,),
        in_specs=[
            pl.BlockSpec((B, H), lambda i: (i, 0)),
            pl.BlockSpec((B, H), lambda i: (i, 0)),
            pl.BlockSpec((B, H), lambda i: (i, 0)),
            pl.BlockSpec((B, 1), lambda i: (i, 0)),
            pl.BlockSpec((1, H), lambda i: (0, 0)),
        ],
        out_specs=pl.BlockSpec((B, H), lambda i: (i, 0)),
        out_shape=jax.ShapeDtypeStruct((N, H), jnp.float32),
    )(a0, a1, hws2, dis, b2)


def _ib_body(bm_ref, ib_ref):
    bm = bm_ref[...]
    gi = lax.broadcasted_iota(jnp.int32, (G, 80, 128), 0)
    m = (bm[None, :, :] < gi).astype(jnp.int32)
    cnt = jnp.sum(jnp.sum(m, axis=2), axis=1)
    ib_ref[...] = cnt[:, None]


def _ib_call(bm):
    return pl.pallas_call(
        _ib_body,
        out_shape=jax.ShapeDtypeStruct((G, 1), jnp.int32),
    )(bm)


def _final_body(mb_ref, wl_ref, bl_ref, out_ref):
    out_ref[...] = jnp.dot(mb_ref[...], wl_ref[...],
                           preferred_element_type=jnp.float32) + bl_ref[...]


def _final_call(mb, Wl, bl2):
    return pl.pallas_call(
        _final_body,
        out_shape=jax.ShapeDtypeStruct((G, OUT), jnp.float32),
    )(mb, Wl, bl2)


@jax.jit
def _run(x, edge_index, set_indice, batch_idx, basis_freq, phase,
         W_ih, W_hh, b_ih, b_hh, W1, b1, W2, b2, Wl, bl):
    f32 = jnp.float32
    i32 = jnp.int32
    src = edge_index[0]
    dst = edge_index[1]
    srcp = jnp.concatenate([src, jnp.zeros((EP - E,), i32)])
    dstp = jnp.concatenate([dst, jnp.full((EP - E,), N, i32)])
    zeros64 = jnp.zeros((RPT, H), f32)
    zeros16 = jnp.zeros((RPT, 16), f32)
    ones16 = jnp.ones((CHUNK, 16), f32)
    bf2 = basis_freq.reshape(1, F)
    ph2 = phase.reshape(1, F)
    wihT = W_ih.T
    whhT = W_hh.T
    bih2 = b_ih.reshape(1, 3 * F)
    bhh2 = b_hh.reshape(1, 3 * F)
    bm = jnp.concatenate([batch_idx,
                          jnp.full((AR - N,), G + 1, i32)]).reshape(80, 128)

    g1 = _gru_call(x, bf2, ph2, wihT, whhT, bih2, bhh2, W1)
    degf = _deg_call(dstp, ones16, zeros16)
    dis, hws1 = _scale_call(degf[:AR], degf[AR:], g1)
    acc1 = _edge_call(hws1, srcp, dstp, zeros64)
    hws2 = _mid_call(acc1[:AR][:N], acc1[AR:][:N], hws1, dis,
                     b1.reshape(1, H), W2)
    acc2 = _edge_call(hws2, srcp, dstp, zeros64)
    h2 = _h2_call(acc2[:AR][:N], acc2[AR:][:N], hws2, dis, b2.reshape(1, H))
    ib = _ib_call(bm)
    sib = jnp.clip(ib + set_indice, 0, N - 1).reshape(-1)
    sibp = jnp.concatenate([sib, jnp.zeros((GB - G * S,), i32)])
    gath = _rgather_call(h2, sibp)
    mb = gath[:G * S].reshape(G, S * H)
    return _final_call(mb, Wl, bl.reshape(1, OUT))


def kernel(x, edge_index, set_indice, batch_idx, num_graphs, basis_freq,
           phase, W_ih, W_hh, b_ih, b_hh, W1, b1, W2, b2, Wl, bl):
    return _run(x, edge_index, set_indice, batch_idx, basis_freq, phase,
                W_ih, W_hh, b_ih, b_hh, W1, b1, W2, b2, Wl, bl)


# speedup vs baseline: 1.8348x; 1.3195x over previous
import functools

import jax
import jax.numpy as jnp
from jax import lax
from jax.experimental import pallas as pl
from jax.experimental.pallas import tpu as pltpu
from jax.experimental.pallas import tpu_sc as plsc

N = 10000
F = 32
H = 64
G = 100
S = 3
OUT = 64
E = 640000

NC = 2
NS = 16
NW = NC * NS
CHUNK = 128
CHUNKS_PER_W = 157
EW = CHUNK * CHUNKS_PER_W
EP = EW * NW
AR = 10240
RPT = AR // NS
GB = 512


def _gru_body(x_ref, bf_ref, ph_ref, wih_ref, whh_ref, bih_ref, bhh_ref,
              w1_ref, out_ref):
    B = x_ref.shape[0]
    xb = x_ref[...]
    bf = bf_ref[...]
    ph = ph_ref[...]
    wih = wih_ref[...]
    whh = whh_ref[...]
    bih = bih_ref[...]
    bhh = bhh_ref[...]
    h = jnp.broadcast_to(jnp.cos(ph), (B, F))
    for t in range(F):
        xt = jnp.cos(xb[:, t:t + 1] * bf + ph)
        gi = jnp.dot(xt, wih, preferred_element_type=jnp.float32) + bih
        gh = jnp.dot(h, whh, preferred_element_type=jnp.float32) + bhh
        r = jax.nn.sigmoid(gi[:, 0:F] + gh[:, 0:F])
        z = jax.nn.sigmoid(gi[:, F:2 * F] + gh[:, F:2 * F])
        ng = jnp.tanh(gi[:, 2 * F:] + r * gh[:, 2 * F:])
        h = (1.0 - z) * ng + z * h
    out_ref[...] = jnp.dot(h, w1_ref[...], preferred_element_type=jnp.float32)


def _gru_call(x, bf2, ph2, wihT, whhT, bih2, bhh2, W1):
    B = 2000
    grid = N // B
    return pl.pallas_call(
        _gru_body,
        grid=(grid,),
        in_specs=[
            pl.BlockSpec((B, F), lambda i: (i, 0)),
            pl.BlockSpec((1, F), lambda i: (0, 0)),
            pl.BlockSpec((1, F), lambda i: (0, 0)),
            pl.BlockSpec((F, 3 * F), lambda i: (0, 0)),
            pl.BlockSpec((F, 3 * F), lambda i: (0, 0)),
            pl.BlockSpec((1, 3 * F), lambda i: (0, 0)),
            pl.BlockSpec((1, 3 * F), lambda i: (0, 0)),
            pl.BlockSpec((F, H), lambda i: (0, 0)),
        ],
        out_specs=pl.BlockSpec((B, H), lambda i: (i, 0)),
        out_shape=jax.ShapeDtypeStruct((N, H), jnp.float32),
    )(x, bf2, ph2, wihT, whhT, bih2, bhh2, W1)


def _deg_body(dst_hbm, ones_hbm, zeros_hbm, out_hbm, onesb, zbuf, didx, acc,
              sem):
    c = lax.axis_index("c")
    s = lax.axis_index("s")
    wid = c * NS + s

    pltpu.sync_copy(zeros_hbm, zbuf)
    pltpu.sync_copy(zbuf, acc.at[pl.ds(s * RPT, RPT)])
    pltpu.sync_copy(ones_hbm, onesb)
    plsc.subcore_barrier()
    base = wid * EW

    def chunk(i, carry):
        pltpu.sync_copy(dst_hbm.at[pl.ds(base + i * CHUNK, CHUNK)], didx)
        pltpu.sync_copy(onesb, acc.at[didx], add=True)
        return carry

    lax.fori_loop(0, CHUNKS_PER_W, chunk, 0)
    plsc.subcore_barrier()
    pltpu.sync_copy(acc.at[pl.ds(s * RPT, RPT)], zbuf)
    pltpu.sync_copy(zbuf, out_hbm.at[pl.ds(c * AR + s * RPT, RPT)])


def _deg_call(dstp, ones16, zeros16):
    mesh = plsc.VectorSubcoreMesh(core_axis_name="c", subcore_axis_name="s")
    kfn = pl.kernel(
        _deg_body,
        mesh=mesh,
        out_type=jax.ShapeDtypeStruct((NC * AR, 16), jnp.float32),
        scratch_types=[
            pltpu.VMEM((CHUNK, 16), jnp.float32),
            pltpu.VMEM((RPT, 16), jnp.float32),
            pltpu.VMEM((CHUNK,), jnp.int32),
            pltpu.VMEM_SHARED((AR, 16), jnp.float32),
            pltpu.SemaphoreType.DMA,
        ],
        compiler_params=pltpu.CompilerParams(use_tc_tiling_on_sc=False),
    )
    return kfn(dstp, ones16, zeros16)


def _edge_body(table_hbm, src_hbm, dst_hbm, zeros_hbm, out_hbm,
               sidx, didx, rows, obuf, acc, sem):
    c = lax.axis_index("c")
    s = lax.axis_index("s")
    wid = c * NS + s

    pltpu.sync_copy(zeros_hbm, obuf)
    pltpu.sync_copy(obuf, acc.at[pl.ds(s * RPT, RPT)])
    base = wid * EW
    plsc.subcore_barrier()

    def chunk(i, carry):
        off = base + i * CHUNK
        pltpu.sync_copy(src_hbm.at[pl.ds(off, CHUNK)], sidx)
        pltpu.sync_copy(dst_hbm.at[pl.ds(off, CHUNK)], didx)
        pltpu.async_copy(table_hbm.at[sidx], rows, sem).wait()
        pltpu.sync_copy(rows, acc.at[didx], add=True)
        return carry

    lax.fori_loop(0, CHUNKS_PER_W, chunk, 0)
    plsc.subcore_barrier()
    pltpu.sync_copy(acc.at[pl.ds(s * RPT, RPT)], obuf)
    pltpu.sync_copy(obuf, out_hbm.at[pl.ds(c * AR + s * RPT, RPT)])


def _edge_call(table, srcp, dstp, zeros64):
    mesh = plsc.VectorSubcoreMesh(core_axis_name="c", subcore_axis_name="s")
    kfn = pl.kernel(
        _edge_body,
        mesh=mesh,
        out_type=jax.ShapeDtypeStruct((NC * AR, H), jnp.float32),
        scratch_types=[
            pltpu.VMEM((CHUNK,), jnp.int32),
            pltpu.VMEM((CHUNK,), jnp.int32),
            pltpu.VMEM((CHUNK, H), jnp.float32),
            pltpu.VMEM((RPT, H), jnp.float32),
            pltpu.VMEM_SHARED((AR, H), jnp.float32),
            pltpu.SemaphoreType.DMA,
        ],
        compiler_params=pltpu.CompilerParams(use_tc_tiling_on_sc=False),
    )
    return kfn(table, srcp, dstp, zeros64)


def _rgather_body(table_hbm, idx_hbm, out_hbm, idxb, rows, sem):
    c = lax.axis_index("c")
    s = lax.axis_index("s")
    wid = c * NS + s
    k = GB // NW
    pltpu.sync_copy(idx_hbm.at[pl.ds(wid * k, k)], idxb)
    pltpu.async_copy(table_hbm.at[idxb], rows, sem).wait()
    pltpu.sync_copy(rows, out_hbm.at[pl.ds(wid * k, k)])


def _rgather_call(h2, sibp):
    mesh = plsc.VectorSubcoreMesh(core_axis_name="c", subcore_axis_name="s")
    kfn = pl.kernel(
        _rgather_body,
        mesh=mesh,
        out_type=jax.ShapeDtypeStruct((GB, H), jnp.float32),
        scratch_types=[
            pltpu.VMEM((GB // NW,), jnp.int32),
            pltpu.VMEM((GB // NW, H), jnp.float32),
            pltpu.SemaphoreType.DMA,
        ],
        compiler_params=pltpu.CompilerParams(use_tc_tiling_on_sc=False),
    )
    return kfn(h2, sibp)


def _scale_body(d0_ref, d1_ref, g1_ref, dis_ref, hws_ref):
    deg = 1.0 + d0_ref[:, 0:1] + d1_ref[:, 0:1]
    dis = lax.rsqrt(deg)
    dis_ref[...] = dis
    hws_ref[...] = g1_ref[...] * dis


def _scale_call(d0, d1, g1):
    B = 1000
    grid = N // B
    return pl.pallas_call(
        _scale_body,
        grid=(grid,),
        in_specs=[
            pl.BlockSpec((B, 16), lambda i: (i, 0)),
            pl.BlockSpec((B, 16), lambda i: (i, 0)),
            pl.BlockSpec((B, H), lambda i: (i, 0)),
        ],
        out_specs=[
            pl.BlockSpec((B, 1), lambda i: (i, 0)),
            pl.BlockSpec((B, H), lambda i: (i, 0)),
        ],
        out_shape=[
            jax.ShapeDtypeStruct((N, 1), jnp.float32),
            jax.ShapeDtypeStruct((N, H), jnp.float32),
        ],
    )(d0, d1, g1)


def _mid_body(a0_ref, a1_ref, hws_ref, dis_ref, b_ref, w_ref, out_ref):
    dis = dis_ref[...]
    h1 = jax.nn.relu((a0_ref[...] + a1_ref[...] + hws_ref[...]) * dis
                     + b_ref[...])
    out_ref[...] = jnp.dot(h1, w_ref[...],
                           preferred_element_type=jnp.float32) * dis


def _mid_call(a0, a1, hws1, dis, b1, W2):
    B = 1000
    grid = N // B
    return pl.pallas_call(
        _mid_body,
        grid=(grid,),
        in_specs=[
            pl.BlockSpec((B, H), lambda i: (i, 0)),
            pl.BlockSpec((B, H), lambda i: (i, 0)),
            pl.BlockSpec((B, H), lambda i: (i, 0)),
            pl.BlockSpec((B, 1), lambda i: (i, 0)),
            pl.BlockSpec((1, H), lambda i: (0, 0)),
            pl.BlockSpec((H, H), lambda i: (0, 0)),
        ],
        out_specs=pl.BlockSpec((B, H), lambda i: (i, 0)),
        out_shape=jax.ShapeDtypeStruct((N, H), jnp.float32),
    )(a0, a1, hws1, dis, b1, W2)


def _h2_body(a0_ref, a1_ref, hws_ref, dis_ref, b_ref, out_ref):
    out_ref[...] = jax.nn.relu(
        (a0_ref[...] + a1_ref[...] + hws_ref[...]) * dis_ref[...] + b_ref[...])


def _h2_call(a0, a1, hws2, dis, b2):
    B = 1000
    grid = N // B
    return pl.pallas_call(
        _h2_body,
        grid=(grid,),
        in_specs=[
            pl.BlockSpec((B, H), lambda i: (i, 0)),
            pl.BlockSpec((B, H), lambda i: (i, 0)),
            pl.BlockSpec((B, H), lambda i: (i, 0)),
            pl.BlockSpec((B, 1), lambda i: (i, 0)),
            pl.BlockSpec((1, H), lambda i: (0, 0)),
        ],
        out_specs=pl.BlockSpec((B, H), lambda i: (i, 0)),
        out_shape=jax.ShapeDtypeStruct((N, H), jnp.float32),
    )(a0, a1, hws2, dis, b2)


def _ib_body(bm_ref, ib_ref):
    bm = bm_ref[...]
    gi = lax.broadcasted_iota(jnp.int32, (G, 80, 128), 0)
    m = (bm[None, :, :] < gi).astype(jnp.int32)
    cnt = jnp.sum(jnp.sum(m, axis=2), axis=1)
    ib_ref[...] = cnt[:, None]


def _ib_call(bm):
    return pl.pallas_call(
        _ib_body,
        out_shape=jax.ShapeDtypeStruct((G, 1), jnp.int32),
    )(bm)


def _final_body(mb_ref, wl_ref, bl_ref, out_ref):
    out_ref[...] = jnp.dot(mb_ref[...], wl_ref[...],
                           preferred_element_type=jnp.float32) + bl_ref[...]


def _final_call(mb, Wl, bl2):
    return pl.pallas_call(
        _final_body,
        out_shape=jax.ShapeDtypeStruct((G, OUT), jnp.float32),
    )(mb, Wl, bl2)


@jax.jit
def _run(x, edge_index, set_indice, batch_idx, basis_freq, phase,
         W_ih, W_hh, b_ih, b_hh, W1, b1, W2, b2, Wl, bl):
    f32 = jnp.float32
    i32 = jnp.int32
    src = edge_index[0]
    dst = edge_index[1]
    srcp = jnp.concatenate([src, jnp.zeros((EP - E,), i32)])
    dstp = jnp.concatenate([dst, jnp.full((EP - E,), N, i32)])
    zeros64 = jnp.zeros((RPT, H), f32)
    zeros16 = jnp.zeros((RPT, 16), f32)
    ones16 = jnp.ones((CHUNK, 16), f32)
    bf2 = basis_freq.reshape(1, F)
    ph2 = phase.reshape(1, F)
    wihT = W_ih.T
    whhT = W_hh.T
    bih2 = b_ih.reshape(1, 3 * F)
    bhh2 = b_hh.reshape(1, 3 * F)
    bm = jnp.concatenate([batch_idx,
                          jnp.full((AR - N,), G + 1, i32)]).reshape(80, 128)

    g1 = _gru_call(x, bf2, ph2, wihT, whhT, bih2, bhh2, W1)
    degf = _deg_call(dstp, ones16, zeros16)
    dis, hws1 = _scale_call(degf[:AR], degf[AR:], g1)
    acc1 = _edge_call(hws1, srcp, dstp, zeros64)
    hws2 = _mid_call(acc1[:AR][:N], acc1[AR:][:N], hws1, dis,
                     b1.reshape(1, H), W2)
    acc2 = _edge_call(hws2, srcp, dstp, zeros64)
    h2 = _h2_call(acc2[:AR][:N], acc2[AR:][:N], hws2, dis, b2.reshape(1, H))
    ib = _ib_call(bm)
    sib = jnp.clip(ib + set_indice, 0, N - 1).reshape(-1)
    sibp = jnp.concatenate([sib, jnp.zeros((GB - G * S,), i32)])
    gath = _rgather_call(h2, sibp)
    mb = gath[:G * S].reshape(G, S * H)
    return _final_call(mb, Wl, bl.reshape(1, OUT))


def kernel(x, edge_index, set_indice, batch_idx, num_graphs, basis_freq,
           phase, W_ih, W_hh, b_ih, b_hh, W1, b1, W2, b2, Wl, bl):
    return _run(x, edge_index, set_indice, batch_idx, basis_freq, phase,
                W_ih, W_hh, b_ih, b_hh, W1, b1, W2, b2, Wl, bl)
